# baseline (speedup 1.0000x reference)
import functools

import jax
import jax.numpy as jnp
import numpy as np
from jax import lax
from jax.experimental import pallas as pl
from jax.experimental.pallas import tpu as pltpu
from jax.experimental.pallas import tpu_sc as plsc

N = 10000
K = 32
H = 128
NW = 32
NPAD = 10240
BPW = NPAD // NW
G = 4
NG = BPW // G
RB = 1024

_COLPERM = np.zeros(H, np.int32)
for _j in range(H // 32):
    for _i in range(16):
        _COLPERM[_j * 32 + 2 * _i] = _j * 32 + _i
        _COLPERM[_j * 32 + 2 * _i + 1] = _j * 32 + 16 + _i


def _score_body(e_ref, w_ref, b_ref, u_ref, o_ref):
    h = jnp.dot(e_ref[...], w_ref[...], preferred_element_type=jnp.float32)
    h = h + b_ref[...]
    h = jnp.where(h >= 0, h, 0.1 * h)
    s = jnp.sum(h * u_ref[...], axis=1)
    o_ref[...] = s.reshape(1, 1, RB)


def _scores_tc(e, wt, b2, u2):
    grid = NPAD // RB
    out = pl.pallas_call(
        _score_body,
        grid=(grid,),
        in_specs=[
            pl.BlockSpec((RB, H), lambda i: (i, 0)),
            pl.BlockSpec((H, H), lambda i: (0, 0)),
            pl.BlockSpec((1, H), lambda i: (0, 0)),
            pl.BlockSpec((1, H), lambda i: (0, 0)),
        ],
        out_specs=pl.BlockSpec((1, 1, RB), lambda i: (i, 0, 0)),
        out_shape=jax.ShapeDtypeStruct((grid, 1, RB), jnp.float32),
    )(e, wt, b2, u2)
    return out.reshape(NPAD)


LASTW = N - (NW - 1) * BPW


def _sc_body(e_hbm, ebf_hbm, nbr_hbm, sc_hbm, out_hbm,
             scores_v, nbr_v, out_v, rows_v, w_v, sem0, sem1):
    wid = lax.axis_index("s") * 2 + lax.axis_index("c")
    base = wid * BPW
    last = wid == NW - 1
    pltpu.sync_copy(sc_hbm, scores_v)
    pltpu.sync_copy(nbr_hbm.at[pl.ds(base * K, BPW * K)], nbr_v)

    @pl.when(jnp.logical_not(last))
    def _():
        pltpu.sync_copy(e_hbm.at[pl.ds(base, BPW)], out_v)

    @pl.when(last)
    def _():
        pltpu.sync_copy(e_hbm.at[pl.ds(N - LASTW, LASTW)],
                        out_v.at[pl.ds(0, LASTW)])

    sems = (sem0, sem1)

    def start_gather(g, buf):
        idx = nbr_v.at[pl.ds(g * (G * K), G * K)]
        pltpu.async_copy(ebf_hbm.at[idx], rows_v.at[buf], sems[buf])

    def compute_group(g, buf):
        idx = nbr_v.at[pl.ds(g * (G * K), G * K)]
        pltpu.make_async_copy(ebf_hbm.at[idx], rows_v.at[buf], sems[buf]).wait()
        for i in range(G):
            node = g * G + i
            i0 = nbr_v[pl.ds(node * K, 16)]
            i1 = nbr_v[pl.ds(node * K + 16, 16)]
            s0 = plsc.load_gather(scores_v, [i0])
            s1 = plsc.load_gather(scores_v, [i1])
            m = jnp.max(jnp.maximum(s0, s1))
            x0 = jnp.exp(s0 - m)
            x1 = jnp.exp(s1 - m)
            tot = lax.broadcast(jnp.sum(x0 + x1), (16,))
            w_v[pl.ds(0, 16)] = x0 / tot
            w_v[pl.ds(16, 16)] = x1 / tot
            acc = [out_v[node, pl.ds(h * 16, 16)] for h in range(8)]
            for k in range(K):
                wk = plsc.load_gather(w_v, [jnp.full((16,), k, jnp.int32)])
                for j in range(H // 32):
                    blk = plsc.bitcast(
                        rows_v[buf, i * K + k, pl.ds(j * 16, 16)],
                        jnp.bfloat16)
                    lo, hi = plsc.unpack(
                        blk, format=plsc.PackFormat.INTERLEAVED)
                    acc[2 * j] = acc[2 * j] + wk * lo
                    acc[2 * j + 1] = acc[2 * j + 1] + wk * hi
            for h in range(8):
                out_v[node, pl.ds(h * 16, 16)] = acc[h]

    start_gather(0, 0)

    def pair(p, carry):
        g0 = 2 * p
        start_gather(g0 + 1, 1)
        compute_group(g0, 0)
        start_gather(jnp.minimum(g0 + 2, NG - 2), 0)
        compute_group(g0 + 1, 1)
        return carry

    lax.fori_loop(0, NG // 2, pair, 0)

    @pl.when(jnp.logical_not(last))
    def _():
        pltpu.sync_copy(out_v, out_hbm.at[pl.ds(base, BPW)])

    @pl.when(last)
    def _():
        pltpu.sync_copy(out_v.at[pl.ds(0, LASTW)],
                        out_hbm.at[pl.ds(N - LASTW, LASTW)])


_sc_aggregate = functools.partial(
    pl.kernel,
    out_type=jax.ShapeDtypeStruct((N, H), jnp.float32),
    mesh=plsc.VectorSubcoreMesh(core_axis_name="c", subcore_axis_name="s"),
    compiler_params=pltpu.CompilerParams(
        needs_layout_passes=False, use_tc_tiling_on_sc=False),
    scratch_types=[
        pltpu.VMEM((NPAD,), jnp.float32),
        pltpu.VMEM((BPW * K,), jnp.int32),
        pltpu.VMEM((BPW, H), jnp.float32),
        pltpu.VMEM((2, G * K, H // 2), jnp.int32),
        pltpu.VMEM((K,), jnp.float32),
        pltpu.SemaphoreType.DMA,
        pltpu.SemaphoreType.DMA,
    ],
)(_sc_body)


@jax.jit
def kernel(e, neighbors, Wa_w, Wa_b, ua):
    e_bfp = lax.bitcast_convert_type(
        e.astype(jnp.bfloat16)[:, _COLPERM].reshape(N, H // 2, 2),
        jnp.int32)
    nbr = jnp.zeros((NPAD, K), jnp.int32).at[:N].set(neighbors.astype(jnp.int32))
    scores = _scores_tc(e, Wa_w.T, Wa_b.reshape(1, H), ua.reshape(1, H))
    return _sc_aggregate(e, e_bfp, nbr.reshape(-1), scores)


# speedup vs baseline: 1.1735x; 1.1735x over previous
import functools

import jax
import jax.numpy as jnp
import numpy as np
from jax import lax
from jax.experimental import pallas as pl
from jax.experimental.pallas import tpu as pltpu
from jax.experimental.pallas import tpu_sc as plsc

N = 10000
K = 32
H = 128
NW = 32
NPAD = 10240
BPW = NPAD // NW
G = 4
NG = BPW // G
RB = 1024

_COLPERM = np.zeros(H, np.int32)
for _j in range(H // 32):
    for _i in range(16):
        _COLPERM[_j * 32 + 2 * _i] = _j * 32 + _i
        _COLPERM[_j * 32 + 2 * _i + 1] = _j * 32 + 16 + _i


def _score_body(e_ref, w_ref, b_ref, u_ref, o_ref):
    h = jnp.dot(e_ref[...], w_ref[...], preferred_element_type=jnp.float32)
    h = h + b_ref[...]
    h = jnp.where(h >= 0, h, 0.1 * h)
    s = jnp.sum(h * u_ref[...], axis=1)
    o_ref[...] = s.reshape(1, 1, RB)


def _scores_tc(e, wt, b2, u2):
    grid = NPAD // RB
    out = pl.pallas_call(
        _score_body,
        grid=(grid,),
        in_specs=[
            pl.BlockSpec((RB, H), lambda i: (i, 0)),
            pl.BlockSpec((H, H), lambda i: (0, 0)),
            pl.BlockSpec((1, H), lambda i: (0, 0)),
            pl.BlockSpec((1, H), lambda i: (0, 0)),
        ],
        out_specs=pl.BlockSpec((1, 1, RB), lambda i: (i, 0, 0)),
        out_shape=jax.ShapeDtypeStruct((grid, 1, RB), jnp.float32),
    )(e, wt, b2, u2)
    return out.reshape(NPAD)


LASTW = N - (NW - 1) * BPW


def _sc_body(e_hbm, ebf_hbm, nbr_hbm, sc_hbm, out_hbm,
             scores_v, nbr_v, out_v, rows_v, w_v, sem0, sem1):
    wid = lax.axis_index("s") * 2 + lax.axis_index("c")
    base = wid * BPW
    last = wid == NW - 1
    pltpu.sync_copy(sc_hbm, scores_v)
    pltpu.sync_copy(nbr_hbm.at[pl.ds(base * K, BPW * K)], nbr_v)

    @pl.when(jnp.logical_not(last))
    def _():
        pltpu.sync_copy(e_hbm.at[pl.ds(base, BPW)], out_v)

    @pl.when(last)
    def _():
        pltpu.sync_copy(e_hbm.at[pl.ds(N - LASTW, LASTW)],
                        out_v.at[pl.ds(0, LASTW)])

    sems = (sem0, sem1)

    def start_gather(g, buf):
        idx = nbr_v.at[pl.ds(g * (G * K), G * K)]
        pltpu.async_copy(ebf_hbm.at[idx], rows_v.at[buf], sems[buf])

    def compute_group(g, buf):
        idx = nbr_v.at[pl.ds(g * (G * K), G * K)]
        pltpu.make_async_copy(ebf_hbm.at[idx], rows_v.at[buf], sems[buf]).wait()
        for i in range(G):
            node = g * G + i
            i0 = nbr_v[pl.ds(node * K, 16)]
            i1 = nbr_v[pl.ds(node * K + 16, 16)]
            s0 = plsc.load_gather(scores_v, [i0])
            s1 = plsc.load_gather(scores_v, [i1])
            m = jnp.max(jnp.maximum(s0, s1))
            x0 = jnp.exp(s0 - m)
            x1 = jnp.exp(s1 - m)
            tot = lax.broadcast(jnp.sum(x0 + x1), (16,))
            w_v[pl.ds(0, 16)] = x0 / tot
            w_v[pl.ds(16, 16)] = x1 / tot
            acc = [out_v[node, pl.ds(h * 16, 16)] for h in range(8)]
            for k in range(K):
                wk = plsc.load_gather(w_v, [jnp.full((16,), k, jnp.int32)])
                for j in range(H // 32):
                    blk = plsc.bitcast(
                        rows_v[buf, i * K + k, pl.ds(j * 16, 16)],
                        jnp.bfloat16)
                    lo, hi = plsc.unpack(
                        blk, format=plsc.PackFormat.INTERLEAVED)
                    acc[2 * j] = acc[2 * j] + wk * lo
                    acc[2 * j + 1] = acc[2 * j + 1] + wk * hi
            for h in range(8):
                out_v[node, pl.ds(h * 16, 16)] = acc[h]

    start_gather(0, 0)

    def pair(p, carry):
        g0 = 2 * p
        start_gather(g0 + 1, 1)
        compute_group(g0, 0)
        start_gather(jnp.minimum(g0 + 2, NG - 2), 0)
        compute_group(g0 + 1, 1)
        return carry

    lax.fori_loop(0, NG // 2, pair, 0)

    @pl.when(jnp.logical_not(last))
    def _():
        pltpu.sync_copy(out_v, out_hbm.at[pl.ds(base, BPW)])

    @pl.when(last)
    def _():
        pltpu.sync_copy(out_v.at[pl.ds(0, LASTW)],
                        out_hbm.at[pl.ds(N - LASTW, LASTW)])


_sc_aggregate = functools.partial(
    pl.kernel,
    out_type=jax.ShapeDtypeStruct((N, H), jnp.float32),
    mesh=plsc.VectorSubcoreMesh(core_axis_name="c", subcore_axis_name="s"),
    compiler_params=pltpu.CompilerParams(
        needs_layout_passes=False, use_tc_tiling_on_sc=False),
    scratch_types=[
        pltpu.VMEM((NPAD,), jnp.float32),
        pltpu.VMEM((BPW * K,), jnp.int32),
        pltpu.VMEM((BPW, H), jnp.float32),
        pltpu.VMEM((2, G * K, H // 2), jnp.int32),
        pltpu.VMEM((K,), jnp.float32),
        pltpu.SemaphoreType.DMA,
        pltpu.SemaphoreType.DMA,
    ],
)(_sc_body)


@jax.jit
def kernel(e, neighbors, Wa_w, Wa_b, ua):
    e_bfp = lax.bitcast_convert_type(
        e.astype(jnp.bfloat16)[:, _COLPERM].reshape(N, H // 2, 2),
        jnp.int32)
    nbr = jnp.zeros((NPAD, K), jnp.int32).at[:N].set(neighbors.astype(jnp.int32))
    e_pad = jnp.zeros((NPAD, H), jnp.float32).at[:N].set(e)
    scores = _scores_tc(e_pad, Wa_w.T, Wa_b.reshape(1, H), ua.reshape(1, H))
    return _sc_aggregate(e, e_bfp, nbr.reshape(-1), scores)


# speedup vs baseline: 1.1756x; 1.0018x over previous
import functools

import jax
import jax.numpy as jnp
import numpy as np
from jax import lax
from jax.experimental import pallas as pl
from jax.experimental.pallas import tpu as pltpu
from jax.experimental.pallas import tpu_sc as plsc

N = 10000
K = 32
H = 128
NW = 32
NPAD = 10240
BPW = NPAD // NW
G = 4
NG = BPW // G
RB = 1024

_COLPERM = np.zeros(H, np.int32)
for _j in range(H // 32):
    for _i in range(16):
        _COLPERM[_j * 32 + 2 * _i] = _j * 32 + _i
        _COLPERM[_j * 32 + 2 * _i + 1] = _j * 32 + 16 + _i


def _score_body(e_ref, w_ref, b_ref, u_ref, o_ref):
    h = jnp.dot(e_ref[...], w_ref[...], preferred_element_type=jnp.float32)
    h = h + b_ref[...]
    h = jnp.where(h >= 0, h, 0.1 * h)
    s = jnp.sum(h * u_ref[...], axis=1)
    o_ref[...] = s.reshape(1, 1, RB)


def _scores_tc(e, wt, b2, u2):
    grid = NPAD // RB
    out = pl.pallas_call(
        _score_body,
        grid=(grid,),
        in_specs=[
            pl.BlockSpec((RB, H), lambda i: (i, 0)),
            pl.BlockSpec((H, H), lambda i: (0, 0)),
            pl.BlockSpec((1, H), lambda i: (0, 0)),
            pl.BlockSpec((1, H), lambda i: (0, 0)),
        ],
        out_specs=pl.BlockSpec((1, 1, RB), lambda i: (i, 0, 0)),
        out_shape=jax.ShapeDtypeStruct((grid, 1, RB), jnp.float32),
    )(e, wt, b2, u2)
    return out.reshape(NPAD)


LASTW = N - (NW - 1) * BPW


def _sc_body(e_hbm, ebf_hbm, nbr_hbm, sc_hbm, out_hbm,
             scores_v, nbr_v, out_v, rows_v, w_v, sem0, sem1, sem2, sem3):
    wid = lax.axis_index("s") * 2 + lax.axis_index("c")
    base = wid * BPW
    last = wid == NW - 1
    pltpu.sync_copy(sc_hbm, scores_v)
    pltpu.sync_copy(nbr_hbm.at[pl.ds(base * K, BPW * K)], nbr_v)

    @pl.when(jnp.logical_not(last))
    def _():
        pltpu.sync_copy(e_hbm.at[pl.ds(base, BPW)], out_v)

    @pl.when(last)
    def _():
        pltpu.sync_copy(e_hbm.at[pl.ds(N - LASTW, LASTW)],
                        out_v.at[pl.ds(0, LASTW)])

    sems = (sem0, sem1, sem2, sem3)

    def start_gather(g, buf):
        idx = nbr_v.at[pl.ds(g * (G * K), G * K)]
        pltpu.async_copy(ebf_hbm.at[idx], rows_v.at[buf], sems[buf])

    def compute_group(g, buf):
        idx = nbr_v.at[pl.ds(g * (G * K), G * K)]
        pltpu.make_async_copy(ebf_hbm.at[idx], rows_v.at[buf], sems[buf]).wait()
        for i in range(G):
            node = g * G + i
            i0 = nbr_v[pl.ds(node * K, 16)]
            i1 = nbr_v[pl.ds(node * K + 16, 16)]
            s0 = plsc.load_gather(scores_v, [i0])
            s1 = plsc.load_gather(scores_v, [i1])
            m = jnp.max(jnp.maximum(s0, s1))
            x0 = jnp.exp(s0 - m)
            x1 = jnp.exp(s1 - m)
            tot = lax.broadcast(jnp.sum(x0 + x1), (16,))
            w_v[pl.ds(0, 16)] = x0 / tot
            w_v[pl.ds(16, 16)] = x1 / tot
            acc = [out_v[node, pl.ds(h * 16, 16)] for h in range(8)]
            for k in range(K):
                wk = plsc.load_gather(w_v, [jnp.full((16,), k, jnp.int32)])
                for j in range(H // 32):
                    blk = plsc.bitcast(
                        rows_v[buf, i * K + k, pl.ds(j * 16, 16)],
                        jnp.bfloat16)
                    lo, hi = plsc.unpack(
                        blk, format=plsc.PackFormat.INTERLEAVED)
                    acc[2 * j] = acc[2 * j] + wk * lo
                    acc[2 * j + 1] = acc[2 * j + 1] + wk * hi
            for h in range(8):
                out_v[node, pl.ds(h * 16, 16)] = acc[h]

    start_gather(0, 0)
    start_gather(1, 1)
    start_gather(2, 2)

    def quad(q, carry):
        g0 = 4 * q
        for b in range(4):
            g_next = g0 + b + 3

            @pl.when(g_next < NG)
            def _():
                start_gather(g_next, (b + 3) % 4)

            compute_group(g0 + b, b)
        return carry

    lax.fori_loop(0, NG // 4, quad, 0)

    @pl.when(jnp.logical_not(last))
    def _():
        pltpu.sync_copy(out_v, out_hbm.at[pl.ds(base, BPW)])

    @pl.when(last)
    def _():
        pltpu.sync_copy(out_v.at[pl.ds(0, LASTW)],
                        out_hbm.at[pl.ds(N - LASTW, LASTW)])


_sc_aggregate = functools.partial(
    pl.kernel,
    out_type=jax.ShapeDtypeStruct((N, H), jnp.float32),
    mesh=plsc.VectorSubcoreMesh(core_axis_name="c", subcore_axis_name="s"),
    compiler_params=pltpu.CompilerParams(
        needs_layout_passes=False, use_tc_tiling_on_sc=False),
    scratch_types=[
        pltpu.VMEM((NPAD,), jnp.float32),
        pltpu.VMEM((BPW * K,), jnp.int32),
        pltpu.VMEM((BPW, H), jnp.float32),
        pltpu.VMEM((4, G * K, H // 2), jnp.int32),
        pltpu.VMEM((K,), jnp.float32),
        pltpu.SemaphoreType.DMA,
        pltpu.SemaphoreType.DMA,
        pltpu.SemaphoreType.DMA,
        pltpu.SemaphoreType.DMA,
    ],
)(_sc_body)


@jax.jit
def kernel(e, neighbors, Wa_w, Wa_b, ua):
    e_bfp = lax.bitcast_convert_type(
        e.astype(jnp.bfloat16)[:, _COLPERM].reshape(N, H // 2, 2),
        jnp.int32)
    nbr = jnp.zeros((NPAD, K), jnp.int32).at[:N].set(neighbors.astype(jnp.int32))
    e_pad = jnp.zeros((NPAD, H), jnp.float32).at[:N].set(e)
    scores = _scores_tc(e_pad, Wa_w.T, Wa_b.reshape(1, H), ua.reshape(1, H))
    return _sc_aggregate(e, e_bfp, nbr.reshape(-1), scores)


# speedup vs baseline: 1.2657x; 1.0766x over previous
import functools

import jax
import jax.numpy as jnp
import numpy as np
from jax import lax
from jax.experimental import pallas as pl
from jax.experimental.pallas import tpu as pltpu
from jax.experimental.pallas import tpu_sc as plsc

N = 10000
K = 32
H = 128
NW = 32
NPAD = 10240
BPW = NPAD // NW
G = 4
NG = BPW // G
RB = 1024

_COLPERM = np.zeros(H, np.int32)
for _j in range(H // 32):
    for _i in range(16):
        _COLPERM[_j * 32 + 2 * _i] = _j * 32 + _i
        _COLPERM[_j * 32 + 2 * _i + 1] = _j * 32 + 16 + _i


def _score_body(e_ref, w_ref, b_ref, u_ref, o_ref):
    h = jnp.dot(e_ref[...], w_ref[...], preferred_element_type=jnp.float32)
    h = h + b_ref[...]
    h = jnp.where(h >= 0, h, 0.1 * h)
    s = jnp.sum(h * u_ref[...], axis=1)
    o_ref[...] = s.reshape(1, 1, RB)


def _scores_tc(e, wt, b2, u2):
    grid = NPAD // RB
    out = pl.pallas_call(
        _score_body,
        grid=(grid,),
        in_specs=[
            pl.BlockSpec((RB, H), lambda i: (i, 0)),
            pl.BlockSpec((H, H), lambda i: (0, 0)),
            pl.BlockSpec((1, H), lambda i: (0, 0)),
            pl.BlockSpec((1, H), lambda i: (0, 0)),
        ],
        out_specs=pl.BlockSpec((1, 1, RB), lambda i: (i, 0, 0)),
        out_shape=jax.ShapeDtypeStruct((grid, 1, RB), jnp.float32),
    )(e, wt, b2, u2)
    return out.reshape(NPAD)


B0 = 384
B1 = 256
OFF1 = 16 * B0
LASTW = N - (OFF1 + 15 * B1)


def _sc_body(e_hbm, ebf_hbm, nbr_hbm, sc_hbm, out_hbm,
             scores_v, nbr_v, out_v, rows_v, w_v, sem0, sem1, sem2, sem3):
    c = lax.axis_index("c")
    s = lax.axis_index("s")
    base = jnp.where(c == 0, s * B0, OFF1 + s * B1)
    ng = jnp.where(c == 0, B0 // G, B1 // G)
    last = jnp.logical_and(c == 1, s == 15)
    pltpu.sync_copy(sc_hbm, scores_v)

    @pl.when(c == 0)
    def _():
        pltpu.sync_copy(nbr_hbm.at[pl.ds(base * K, B0 * K)], nbr_v)
        pltpu.sync_copy(e_hbm.at[pl.ds(base, B0)], out_v)

    @pl.when(jnp.logical_and(c == 1, s < 15))
    def _():
        pltpu.sync_copy(nbr_hbm.at[pl.ds(base * K, B1 * K)],
                        nbr_v.at[pl.ds(0, B1 * K)])
        pltpu.sync_copy(e_hbm.at[pl.ds(base, B1)], out_v.at[pl.ds(0, B1)])

    @pl.when(last)
    def _():
        pltpu.sync_copy(nbr_hbm.at[pl.ds(base * K, B1 * K)],
                        nbr_v.at[pl.ds(0, B1 * K)])
        pltpu.sync_copy(e_hbm.at[pl.ds(base, LASTW)],
                        out_v.at[pl.ds(0, LASTW)])

    sems = (sem0, sem1, sem2, sem3)

    def start_gather(g, buf):
        idx = nbr_v.at[pl.ds(g * (G * K), G * K)]
        pltpu.async_copy(ebf_hbm.at[idx], rows_v.at[buf], sems[buf])

    def compute_group(g, buf):
        idx = nbr_v.at[pl.ds(g * (G * K), G * K)]
        pltpu.make_async_copy(ebf_hbm.at[idx], rows_v.at[buf], sems[buf]).wait()
        for i in range(G):
            node = g * G + i
            i0 = nbr_v[pl.ds(node * K, 16)]
            i1 = nbr_v[pl.ds(node * K + 16, 16)]
            s0 = plsc.load_gather(scores_v, [i0])
            s1 = plsc.load_gather(scores_v, [i1])
            m = jnp.max(jnp.maximum(s0, s1))
            x0 = jnp.exp(s0 - m)
            x1 = jnp.exp(s1 - m)
            tot = lax.broadcast(jnp.sum(x0 + x1), (16,))
            w_v[pl.ds(0, 16)] = x0 / tot
            w_v[pl.ds(16, 16)] = x1 / tot
            acc = [out_v[node, pl.ds(h * 16, 16)] for h in range(8)]
            for k in range(K):
                wk = plsc.load_gather(w_v, [jnp.full((16,), k, jnp.int32)])
                for j in range(H // 32):
                    blk = plsc.bitcast(
                        rows_v[buf, i * K + k, pl.ds(j * 16, 16)],
                        jnp.bfloat16)
                    lo, hi = plsc.unpack(
                        blk, format=plsc.PackFormat.INTERLEAVED)
                    acc[2 * j] = acc[2 * j] + wk * lo
                    acc[2 * j + 1] = acc[2 * j + 1] + wk * hi
            for h in range(8):
                out_v[node, pl.ds(h * 16, 16)] = acc[h]

    start_gather(0, 0)
    start_gather(1, 1)
    start_gather(2, 2)

    def quad(q, carry):
        g0 = 4 * q
        for b in range(4):
            g_next = g0 + b + 3

            @pl.when(g_next < ng)
            def _():
                start_gather(g_next, (b + 3) % 4)

            compute_group(g0 + b, b)
        return carry

    lax.fori_loop(0, ng // 4, quad, 0)

    @pl.when(c == 0)
    def _():
        pltpu.sync_copy(out_v, out_hbm.at[pl.ds(base, B0)])

    @pl.when(jnp.logical_and(c == 1, s < 15))
    def _():
        pltpu.sync_copy(out_v.at[pl.ds(0, B1)], out_hbm.at[pl.ds(base, B1)])

    @pl.when(last)
    def _():
        pltpu.sync_copy(out_v.at[pl.ds(0, LASTW)],
                        out_hbm.at[pl.ds(base, LASTW)])


_sc_aggregate = functools.partial(
    pl.kernel,
    out_type=jax.ShapeDtypeStruct((N, H), jnp.float32),
    mesh=plsc.VectorSubcoreMesh(core_axis_name="c", subcore_axis_name="s"),
    compiler_params=pltpu.CompilerParams(
        needs_layout_passes=False, use_tc_tiling_on_sc=False),
    scratch_types=[
        pltpu.VMEM((NPAD,), jnp.float32),
        pltpu.VMEM((B0 * K,), jnp.int32),
        pltpu.VMEM((B0, H), jnp.float32),
        pltpu.VMEM((4, G * K, H // 2), jnp.int32),
        pltpu.VMEM((K,), jnp.float32),
        pltpu.SemaphoreType.DMA,
        pltpu.SemaphoreType.DMA,
        pltpu.SemaphoreType.DMA,
        pltpu.SemaphoreType.DMA,
    ],
)(_sc_body)


@jax.jit
def kernel(e, neighbors, Wa_w, Wa_b, ua):
    e_bfp = lax.bitcast_convert_type(
        e.astype(jnp.bfloat16)[:, _COLPERM].reshape(N, H // 2, 2),
        jnp.int32)
    nbr = jnp.zeros((NPAD, K), jnp.int32).at[:N].set(neighbors.astype(jnp.int32))
    e_pad = jnp.zeros((NPAD, H), jnp.float32).at[:N].set(e)
    scores = _scores_tc(e_pad, Wa_w.T, Wa_b.reshape(1, H), ua.reshape(1, H))
    return _sc_aggregate(e, e_bfp, nbr.reshape(-1), scores)


# speedup vs baseline: 1.2751x; 1.0075x over previous
import functools

import jax
import jax.numpy as jnp
import numpy as np
from jax import lax
from jax.experimental import pallas as pl
from jax.experimental.pallas import tpu as pltpu
from jax.experimental.pallas import tpu_sc as plsc

N = 10000
K = 32
H = 128
NW = 32
NPAD = 10240
BPW = NPAD // NW
G = 4
NG = BPW // G
RB = 1024

_COLPERM = np.zeros(H, np.int32)
for _j in range(H // 32):
    for _i in range(16):
        _COLPERM[_j * 32 + 2 * _i] = _j * 32 + _i
        _COLPERM[_j * 32 + 2 * _i + 1] = _j * 32 + 16 + _i


def _score_body(e_ref, w_ref, b_ref, u_ref, o_ref):
    h = jnp.dot(e_ref[...], w_ref[...], preferred_element_type=jnp.float32)
    h = h + b_ref[...]
    h = jnp.where(h >= 0, h, 0.1 * h)
    s = jnp.sum(h * u_ref[...], axis=1)
    o_ref[...] = s.reshape(1, 1, RB)


def _scores_tc(e, wt, b2, u2):
    grid = NPAD // RB
    out = pl.pallas_call(
        _score_body,
        grid=(grid,),
        in_specs=[
            pl.BlockSpec((RB, H), lambda i: (i, 0)),
            pl.BlockSpec((H, H), lambda i: (0, 0)),
            pl.BlockSpec((1, H), lambda i: (0, 0)),
            pl.BlockSpec((1, H), lambda i: (0, 0)),
        ],
        out_specs=pl.BlockSpec((1, 1, RB), lambda i: (i, 0, 0)),
        out_shape=jax.ShapeDtypeStruct((grid, 1, RB), jnp.float32),
    )(e, wt, b2, u2)
    return out.reshape(NPAD)


B0 = 400
B1 = 240
OFF1 = 16 * B0
LASTW = N - (OFF1 + 15 * B1)


def _sc_body(e_hbm, ebf_hbm, nbr_hbm, sc_hbm, out_hbm,
             scores_v, nbr_v, out_v, rows_v, w_v, sem0, sem1, sem2, sem3):
    c = lax.axis_index("c")
    s = lax.axis_index("s")
    base = jnp.where(c == 0, s * B0, OFF1 + s * B1)
    ng = jnp.where(c == 0, B0 // G, B1 // G)
    last = jnp.logical_and(c == 1, s == 15)
    if LASTW == 0:
        ng = jnp.where(last, 0, ng)
    pltpu.sync_copy(sc_hbm, scores_v)

    @pl.when(c == 0)
    def _():
        pltpu.sync_copy(nbr_hbm.at[pl.ds(base * K, B0 * K)], nbr_v)
        pltpu.sync_copy(e_hbm.at[pl.ds(base, B0)], out_v)

    @pl.when(jnp.logical_and(c == 1, s < 15))
    def _():
        pltpu.sync_copy(nbr_hbm.at[pl.ds(base * K, B1 * K)],
                        nbr_v.at[pl.ds(0, B1 * K)])
        pltpu.sync_copy(e_hbm.at[pl.ds(base, B1)], out_v.at[pl.ds(0, B1)])

    if LASTW > 0:
        @pl.when(last)
        def _():
            pltpu.sync_copy(nbr_hbm.at[pl.ds(base * K, B1 * K)],
                            nbr_v.at[pl.ds(0, B1 * K)])
            pltpu.sync_copy(e_hbm.at[pl.ds(base, LASTW)],
                            out_v.at[pl.ds(0, LASTW)])

    sems = (sem0, sem1, sem2, sem3)

    def start_gather(g, buf):
        idx = nbr_v.at[pl.ds(g * (G * K), G * K)]
        pltpu.async_copy(ebf_hbm.at[idx], rows_v.at[buf], sems[buf])

    def compute_group(g, buf):
        idx = nbr_v.at[pl.ds(g * (G * K), G * K)]
        pltpu.make_async_copy(ebf_hbm.at[idx], rows_v.at[buf], sems[buf]).wait()
        for i in range(G):
            node = g * G + i
            i0 = nbr_v[pl.ds(node * K, 16)]
            i1 = nbr_v[pl.ds(node * K + 16, 16)]
            s0 = plsc.load_gather(scores_v, [i0])
            s1 = plsc.load_gather(scores_v, [i1])
            m = jnp.max(jnp.maximum(s0, s1))
            x0 = jnp.exp(s0 - m)
            x1 = jnp.exp(s1 - m)
            tot = lax.broadcast(jnp.sum(x0 + x1), (16,))
            w_v[pl.ds(0, 16)] = x0 / tot
            w_v[pl.ds(16, 16)] = x1 / tot
            acc = [out_v[node, pl.ds(h * 16, 16)] for h in range(8)]
            for k in range(K):
                wk = plsc.load_gather(w_v, [jnp.full((16,), k, jnp.int32)])
                for j in range(H // 32):
                    blk = plsc.bitcast(
                        rows_v[buf, i * K + k, pl.ds(j * 16, 16)],
                        jnp.bfloat16)
                    lo, hi = plsc.unpack(
                        blk, format=plsc.PackFormat.INTERLEAVED)
                    acc[2 * j] = acc[2 * j] + wk * lo
                    acc[2 * j + 1] = acc[2 * j + 1] + wk * hi
            for h in range(8):
                out_v[node, pl.ds(h * 16, 16)] = acc[h]

    for _g in range(3):
        @pl.when(_g < ng)
        def _(_g=_g):
            start_gather(_g, _g)

    def quad(q, carry):
        g0 = 4 * q
        for b in range(4):
            g_next = g0 + b + 3

            @pl.when(g_next < ng)
            def _():
                start_gather(g_next, (b + 3) % 4)

            compute_group(g0 + b, b)
        return carry

    lax.fori_loop(0, ng // 4, quad, 0)

    @pl.when(c == 0)
    def _():
        pltpu.sync_copy(out_v, out_hbm.at[pl.ds(base, B0)])

    @pl.when(jnp.logical_and(c == 1, s < 15))
    def _():
        pltpu.sync_copy(out_v.at[pl.ds(0, B1)], out_hbm.at[pl.ds(base, B1)])

    if LASTW > 0:
        @pl.when(last)
        def _():
            pltpu.sync_copy(out_v.at[pl.ds(0, LASTW)],
                            out_hbm.at[pl.ds(base, LASTW)])


_sc_aggregate = functools.partial(
    pl.kernel,
    out_type=jax.ShapeDtypeStruct((N, H), jnp.float32),
    mesh=plsc.VectorSubcoreMesh(core_axis_name="c", subcore_axis_name="s"),
    compiler_params=pltpu.CompilerParams(
        needs_layout_passes=False, use_tc_tiling_on_sc=False),
    scratch_types=[
        pltpu.VMEM((NPAD,), jnp.float32),
        pltpu.VMEM((B0 * K,), jnp.int32),
        pltpu.VMEM((B0, H), jnp.float32),
        pltpu.VMEM((4, G * K, H // 2), jnp.int32),
        pltpu.VMEM((K,), jnp.float32),
        pltpu.SemaphoreType.DMA,
        pltpu.SemaphoreType.DMA,
        pltpu.SemaphoreType.DMA,
        pltpu.SemaphoreType.DMA,
    ],
)(_sc_body)


@jax.jit
def kernel(e, neighbors, Wa_w, Wa_b, ua):
    e_bfp = lax.bitcast_convert_type(
        e.astype(jnp.bfloat16)[:, _COLPERM].reshape(N, H // 2, 2),
        jnp.int32)
    nbr = jnp.zeros((NPAD, K), jnp.int32).at[:N].set(neighbors.astype(jnp.int32))
    e_pad = jnp.zeros((NPAD, H), jnp.float32).at[:N].set(e)
    scores = _scores_tc(e_pad, Wa_w.T, Wa_b.reshape(1, H), ua.reshape(1, H))
    return _sc_aggregate(e, e_bfp, nbr.reshape(-1), scores)


# speedup vs baseline: 1.3742x; 1.0776x over previous
import functools

import jax
import jax.numpy as jnp
import numpy as np
from jax import lax
from jax.experimental import pallas as pl
from jax.experimental.pallas import tpu as pltpu
from jax.experimental.pallas import tpu_sc as plsc

N = 10000
K = 32
H = 128
NW = 32
NPAD = 10240
BPW = NPAD // NW
G = 4
NG = BPW // G
RB = 1024

_COLPERM = np.zeros(H, np.int32)
for _j in range(H // 32):
    for _i in range(16):
        _COLPERM[_j * 32 + 2 * _i] = _j * 32 + _i
        _COLPERM[_j * 32 + 2 * _i + 1] = _j * 32 + 16 + _i


def _score_body(e_ref, w_ref, b_ref, u_ref, o_ref):
    h = jnp.dot(e_ref[...], w_ref[...], preferred_element_type=jnp.float32)
    h = h + b_ref[...]
    h = jnp.where(h >= 0, h, 0.1 * h)
    s = jnp.sum(h * u_ref[...], axis=1)
    o_ref[...] = s.reshape(1, 1, RB)


def _scores_tc(e, wt, b2, u2):
    grid = NPAD // RB
    out = pl.pallas_call(
        _score_body,
        grid=(grid,),
        in_specs=[
            pl.BlockSpec((RB, H), lambda i: (i, 0)),
            pl.BlockSpec((H, H), lambda i: (0, 0)),
            pl.BlockSpec((1, H), lambda i: (0, 0)),
            pl.BlockSpec((1, H), lambda i: (0, 0)),
        ],
        out_specs=pl.BlockSpec((1, 1, RB), lambda i: (i, 0, 0)),
        out_shape=jax.ShapeDtypeStruct((grid, 1, RB), jnp.float32),
    )(e, wt, b2, u2)
    return out.reshape(NPAD)


B0 = 400
B1 = 240
OFF1 = 16 * B0
LASTW = N - (OFF1 + 15 * B1)


def _sc_body(e_hbm, ebf_hbm, nbr_hbm, sc_hbm, out_hbm,
             scores_v, nbr_v, out_v, rows_v, w_v, sem0, sem1, sem2, sem3):
    c = lax.axis_index("c")
    s = lax.axis_index("s")
    base = jnp.where(c == 0, s * B0, OFF1 + s * B1)
    ng = jnp.where(c == 0, B0 // G, B1 // G)
    last = jnp.logical_and(c == 1, s == 15)
    if LASTW == 0:
        ng = jnp.where(last, 0, ng)
    pltpu.sync_copy(sc_hbm, scores_v)

    @pl.when(c == 0)
    def _():
        pltpu.sync_copy(nbr_hbm.at[pl.ds(base * K, B0 * K)], nbr_v)
        pltpu.sync_copy(e_hbm.at[pl.ds(base, B0)], out_v)

    @pl.when(jnp.logical_and(c == 1, s < 15))
    def _():
        pltpu.sync_copy(nbr_hbm.at[pl.ds(base * K, B1 * K)],
                        nbr_v.at[pl.ds(0, B1 * K)])
        pltpu.sync_copy(e_hbm.at[pl.ds(base, B1)], out_v.at[pl.ds(0, B1)])

    if LASTW > 0:
        @pl.when(last)
        def _():
            pltpu.sync_copy(nbr_hbm.at[pl.ds(base * K, B1 * K)],
                            nbr_v.at[pl.ds(0, B1 * K)])
            pltpu.sync_copy(e_hbm.at[pl.ds(base, LASTW)],
                            out_v.at[pl.ds(0, LASTW)])

    sems = (sem0, sem1, sem2, sem3)

    def start_gather(g, buf):
        idx = nbr_v.at[pl.ds(g * (G * K), G * K)]
        pltpu.async_copy(ebf_hbm.at[idx], rows_v.at[buf], sems[buf])

    def compute_group(g, buf):
        idx = nbr_v.at[pl.ds(g * (G * K), G * K)]
        pltpu.make_async_copy(ebf_hbm.at[idx], rows_v.at[buf], sems[buf]).wait()
        for i in range(G):
            node = g * G + i
            i0 = nbr_v[pl.ds(node * K, 16)]
            i1 = nbr_v[pl.ds(node * K + 16, 16)]
            s0 = plsc.load_gather(scores_v, [i0])
            s1 = plsc.load_gather(scores_v, [i1])
            m = jnp.max(jnp.maximum(s0, s1))
            x0 = jnp.exp(s0 - m)
            x1 = jnp.exp(s1 - m)
            tot = lax.broadcast(jnp.sum(x0 + x1), (16,))
            w_v[pl.ds(0, 16)] = x0 / tot
            w_v[pl.ds(16, 16)] = x1 / tot
            acc = [out_v[node, pl.ds(h * 16, 16)] for h in range(8)]
            for k in range(K):
                wk = plsc.load_gather(w_v, [jnp.full((16,), k, jnp.int32)])
                wk_bf = plsc.pack(wk, wk, format=plsc.PackFormat.INTERLEAVED)
                for j in range(H // 32):
                    blk = plsc.bitcast(
                        rows_v[buf, i * K + k, pl.ds(j * 16, 16)],
                        jnp.bfloat16)
                    lo, hi = plsc.unpack(
                        wk_bf * blk, format=plsc.PackFormat.INTERLEAVED)
                    acc[2 * j] = acc[2 * j] + lo
                    acc[2 * j + 1] = acc[2 * j + 1] + hi
            for h in range(8):
                out_v[node, pl.ds(h * 16, 16)] = acc[h]

    for _g in range(3):
        @pl.when(_g < ng)
        def _(_g=_g):
            start_gather(_g, _g)

    def quad(q, carry):
        g0 = 4 * q
        for b in range(4):
            g_next = g0 + b + 3

            @pl.when(g_next < ng)
            def _():
                start_gather(g_next, (b + 3) % 4)

            compute_group(g0 + b, b)
        return carry

    lax.fori_loop(0, ng // 4, quad, 0)

    @pl.when(c == 0)
    def _():
        pltpu.sync_copy(out_v, out_hbm.at[pl.ds(base, B0)])

    @pl.when(jnp.logical_and(c == 1, s < 15))
    def _():
        pltpu.sync_copy(out_v.at[pl.ds(0, B1)], out_hbm.at[pl.ds(base, B1)])

    if LASTW > 0:
        @pl.when(last)
        def _():
            pltpu.sync_copy(out_v.at[pl.ds(0, LASTW)],
                            out_hbm.at[pl.ds(base, LASTW)])


_sc_aggregate = functools.partial(
    pl.kernel,
    out_type=jax.ShapeDtypeStruct((N, H), jnp.float32),
    mesh=plsc.VectorSubcoreMesh(core_axis_name="c", subcore_axis_name="s"),
    compiler_params=pltpu.CompilerParams(
        needs_layout_passes=False, use_tc_tiling_on_sc=False),
    scratch_types=[
        pltpu.VMEM((NPAD,), jnp.float32),
        pltpu.VMEM((B0 * K,), jnp.int32),
        pltpu.VMEM((B0, H), jnp.float32),
        pltpu.VMEM((4, G * K, H // 2), jnp.int32),
        pltpu.VMEM((K,), jnp.float32),
        pltpu.SemaphoreType.DMA,
        pltpu.SemaphoreType.DMA,
        pltpu.SemaphoreType.DMA,
        pltpu.SemaphoreType.DMA,
    ],
)(_sc_body)


@jax.jit
def kernel(e, neighbors, Wa_w, Wa_b, ua):
    e_bfp = lax.bitcast_convert_type(
        e.astype(jnp.bfloat16)[:, _COLPERM].reshape(N, H // 2, 2),
        jnp.int32)
    nbr = jnp.zeros((NPAD, K), jnp.int32).at[:N].set(neighbors.astype(jnp.int32))
    e_pad = jnp.zeros((NPAD, H), jnp.float32).at[:N].set(e)
    scores = _scores_tc(e_pad, Wa_w.T, Wa_b.reshape(1, H), ua.reshape(1, H))
    return _sc_aggregate(e, e_bfp, nbr.reshape(-1), scores)


# speedup vs baseline: 1.5133x; 1.1013x over previous
import functools

import jax
import jax.numpy as jnp
import numpy as np
from jax import lax
from jax.experimental import pallas as pl
from jax.experimental.pallas import tpu as pltpu
from jax.experimental.pallas import tpu_sc as plsc

N = 10000
K = 32
H = 128
NW = 32
NPAD = 10240
BPW = NPAD // NW
G = 4
NG = BPW // G
RB = 1024

def _pack_rows(e):
    a = e.reshape(N, H // 32, 2, 16)
    u16 = lax.bitcast_convert_type(a.astype(jnp.bfloat16), jnp.uint16)
    lo = u16[:, :, 0, :].astype(jnp.uint32)
    hi = u16[:, :, 1, :].astype(jnp.uint32)
    word = lo | (hi << 16)
    return lax.bitcast_convert_type(word, jnp.int32).reshape(N, H // 2)


def _score_body(e_ref, w_ref, b_ref, u_ref, o_ref):
    h = jnp.dot(e_ref[...], w_ref[...], preferred_element_type=jnp.float32)
    h = h + b_ref[...]
    h = jnp.where(h >= 0, h, 0.1 * h)
    s = jnp.sum(h * u_ref[...], axis=1)
    o_ref[...] = s.reshape(1, 1, RB)


def _scores_tc(e, wt, b2, u2):
    grid = NPAD // RB
    out = pl.pallas_call(
        _score_body,
        grid=(grid,),
        in_specs=[
            pl.BlockSpec((RB, H), lambda i: (i, 0)),
            pl.BlockSpec((H, H), lambda i: (0, 0)),
            pl.BlockSpec((1, H), lambda i: (0, 0)),
            pl.BlockSpec((1, H), lambda i: (0, 0)),
        ],
        out_specs=pl.BlockSpec((1, 1, RB), lambda i: (i, 0, 0)),
        out_shape=jax.ShapeDtypeStruct((grid, 1, RB), jnp.float32),
    )(e, wt, b2, u2)
    return out.reshape(NPAD)


B0 = 400
B1 = 240
OFF1 = 16 * B0
LASTW = N - (OFF1 + 15 * B1)


def _sc_body(e_hbm, ebf_hbm, nbr_hbm, sc_hbm, out_hbm,
             scores_v, nbr_v, out_v, rows_v, w_v, sem0, sem1, sem2, sem3):
    c = lax.axis_index("c")
    s = lax.axis_index("s")
    base = jnp.where(c == 0, s * B0, OFF1 + s * B1)
    ng = jnp.where(c == 0, B0 // G, B1 // G)
    last = jnp.logical_and(c == 1, s == 15)
    if LASTW == 0:
        ng = jnp.where(last, 0, ng)
    pltpu.sync_copy(sc_hbm, scores_v)

    @pl.when(c == 0)
    def _():
        pltpu.sync_copy(nbr_hbm.at[pl.ds(base * K, B0 * K)], nbr_v)
        pltpu.sync_copy(e_hbm.at[pl.ds(base, B0)], out_v)

    @pl.when(jnp.logical_and(c == 1, s < 15))
    def _():
        pltpu.sync_copy(nbr_hbm.at[pl.ds(base * K, B1 * K)],
                        nbr_v.at[pl.ds(0, B1 * K)])
        pltpu.sync_copy(e_hbm.at[pl.ds(base, B1)], out_v.at[pl.ds(0, B1)])

    if LASTW > 0:
        @pl.when(last)
        def _():
            pltpu.sync_copy(nbr_hbm.at[pl.ds(base * K, B1 * K)],
                            nbr_v.at[pl.ds(0, B1 * K)])
            pltpu.sync_copy(e_hbm.at[pl.ds(base, LASTW)],
                            out_v.at[pl.ds(0, LASTW)])

    sems = (sem0, sem1, sem2, sem3)

    def start_gather(g, buf):
        idx = nbr_v.at[pl.ds(g * (G * K), G * K)]
        pltpu.async_copy(ebf_hbm.at[idx], rows_v.at[buf], sems[buf])

    def compute_group(g, buf):
        idx = nbr_v.at[pl.ds(g * (G * K), G * K)]
        pltpu.make_async_copy(ebf_hbm.at[idx], rows_v.at[buf], sems[buf]).wait()
        for i in range(G):
            node = g * G + i
            i0 = nbr_v[pl.ds(node * K, 16)]
            i1 = nbr_v[pl.ds(node * K + 16, 16)]
            s0 = plsc.load_gather(scores_v, [i0])
            s1 = plsc.load_gather(scores_v, [i1])
            m = jnp.max(jnp.maximum(s0, s1))
            x0 = jnp.exp(s0 - m)
            x1 = jnp.exp(s1 - m)
            tot = lax.broadcast(jnp.sum(x0 + x1), (16,))
            w_v[pl.ds(0, 16)] = x0 / tot
            w_v[pl.ds(16, 16)] = x1 / tot
            acc = [out_v[node, pl.ds(h * 16, 16)] for h in range(8)]
            for k in range(K):
                wk = plsc.load_gather(w_v, [jnp.full((16,), k, jnp.int32)])
                wk_bf = plsc.pack(wk, wk, format=plsc.PackFormat.INTERLEAVED)
                for j in range(H // 32):
                    blk = plsc.bitcast(
                        rows_v[buf, i * K + k, pl.ds(j * 16, 16)],
                        jnp.bfloat16)
                    lo, hi = plsc.unpack(
                        wk_bf * blk, format=plsc.PackFormat.INTERLEAVED)
                    acc[2 * j] = acc[2 * j] + lo
                    acc[2 * j + 1] = acc[2 * j + 1] + hi
            for h in range(8):
                out_v[node, pl.ds(h * 16, 16)] = acc[h]

    for _g in range(3):
        @pl.when(_g < ng)
        def _(_g=_g):
            start_gather(_g, _g)

    def quad(q, carry):
        g0 = 4 * q
        for b in range(4):
            g_next = g0 + b + 3

            @pl.when(g_next < ng)
            def _():
                start_gather(g_next, (b + 3) % 4)

            compute_group(g0 + b, b)
        return carry

    lax.fori_loop(0, ng // 4, quad, 0)

    @pl.when(c == 0)
    def _():
        pltpu.sync_copy(out_v, out_hbm.at[pl.ds(base, B0)])

    @pl.when(jnp.logical_and(c == 1, s < 15))
    def _():
        pltpu.sync_copy(out_v.at[pl.ds(0, B1)], out_hbm.at[pl.ds(base, B1)])

    if LASTW > 0:
        @pl.when(last)
        def _():
            pltpu.sync_copy(out_v.at[pl.ds(0, LASTW)],
                            out_hbm.at[pl.ds(base, LASTW)])


_sc_aggregate = functools.partial(
    pl.kernel,
    out_type=jax.ShapeDtypeStruct((N, H), jnp.float32),
    mesh=plsc.VectorSubcoreMesh(core_axis_name="c", subcore_axis_name="s"),
    compiler_params=pltpu.CompilerParams(
        needs_layout_passes=False, use_tc_tiling_on_sc=False),
    scratch_types=[
        pltpu.VMEM((NPAD,), jnp.float32),
        pltpu.VMEM((B0 * K,), jnp.int32),
        pltpu.VMEM((B0, H), jnp.float32),
        pltpu.VMEM((4, G * K, H // 2), jnp.int32),
        pltpu.VMEM((K,), jnp.float32),
        pltpu.SemaphoreType.DMA,
        pltpu.SemaphoreType.DMA,
        pltpu.SemaphoreType.DMA,
        pltpu.SemaphoreType.DMA,
    ],
)(_sc_body)


@jax.jit
def kernel(e, neighbors, Wa_w, Wa_b, ua):
    e_bfp = _pack_rows(e)
    nbr = jnp.zeros((NPAD, K), jnp.int32).at[:N].set(neighbors.astype(jnp.int32))
    e_pad = jnp.zeros((NPAD, H), jnp.float32).at[:N].set(e)
    scores = _scores_tc(e_pad, Wa_w.T, Wa_b.reshape(1, H), ua.reshape(1, H))
    return _sc_aggregate(e, e_bfp, nbr.reshape(-1), scores)
